# Initial kernel scaffold; baseline (speedup 1.0000x reference)
import jax
import jax.numpy as jnp
from jax.experimental import pallas as pl


def kernel(labels, log_probs):
    raise NotImplementedError("write your pallas kernel here")


# speedup vs baseline: 1.5653x; 1.5653x over previous
import functools

import jax
import jax.numpy as jnp
from jax import lax
from jax.experimental import pallas as pl
from jax.experimental.pallas import tpu as pltpu
from jax.experimental.pallas import tpu_sc as plsc

_NUM_CORES = 2
_NUM_SUBCORES = 16
_NW = _NUM_CORES * _NUM_SUBCORES
_CHUNK = 128


@functools.partial(jax.jit, static_argnums=())
def _gather(labels_r, log_probs):
    nw, n_ch, ch = labels_r.shape
    _, d = log_probs.shape
    mesh = plsc.VectorSubcoreMesh(core_axis_name="c", subcore_axis_name="s")

    @functools.partial(
        pl.kernel,
        mesh=mesh,
        out_type=jax.ShapeDtypeStruct((nw, n_ch, ch, d), jnp.float32),
        scratch_types=[
            pltpu.VMEM((n_ch, ch), jnp.int32),
            pltpu.VMEM((n_ch, ch, d), jnp.float32),
            pltpu.SemaphoreType.DMA,
        ],
    )
    def body(labels_hbm, table_hbm, out_hbm, idx_v, rows_v, sem):
        wid = lax.axis_index("s") * _NUM_CORES + lax.axis_index("c")
        pltpu.sync_copy(labels_hbm.at[wid], idx_v)
        copies = [
            pltpu.async_copy(table_hbm.at[idx_v.at[j]], rows_v.at[j], sem)
            for j in range(n_ch)
        ]
        for c in copies:
            c.wait()
        pltpu.sync_copy(rows_v, out_hbm.at[wid])

    return body(labels_r, log_probs)


def kernel(labels, log_probs):
    (b,) = labels.shape
    _, d = log_probs.shape
    b_per_w = b // _NW
    n_ch = b_per_w // _CHUNK
    labels_r = labels.astype(jnp.int32).reshape(_NW, n_ch, _CHUNK)
    out = _gather(labels_r, log_probs)
    return out.reshape(b, d)
